# baseline (speedup 1.0000x reference)
import dataclasses
import functools

import jax
import jax.numpy as jnp
from jax import lax
from jax.experimental import pallas as pl
from jax.experimental.pallas import tpu as pltpu
from jax.experimental.pallas import tpu_sc as plsc

_NC = 2
_NS = 16
_L = 16
_SW = 64
_CHW = 8


def _sc_params():
    cp = pltpu.CompilerParams()
    for f, v in (("needs_layout_passes", False), ("use_tc_tiling_on_sc", False)):
        if f in pltpu.CompilerParams.__dataclass_fields__:
            cp = dataclasses.replace(cp, **{f: v})
    return cp


def _sc_aggregate(xq, src2, dst2, w2, n_pad, n_feat):
    D = n_feat
    WN = src2.shape[0]
    NP = n_pad
    RPS = NP // _NS
    NW = _NC * _NS
    DH = D // 2
    DB = D // 32

    W3 = WN // NW
    W1 = WN // _NS
    NCH3 = W3 // _CHW
    NCH1 = W1 // _CHW
    assert WN % NW == 0 and W3 % _CHW == 0 and W1 % _CHW == 0
    assert RPS % _SW == 0

    mesh = plsc.VectorSubcoreMesh(core_axis_name="c", subcore_axis_name="s")

    @functools.partial(
        pl.kernel,
        out_type=(
            jax.ShapeDtypeStruct((_NC, NP, D), jnp.float32),
            jax.ShapeDtypeStruct((NP,), jnp.float32),
        ),
        mesh=mesh,
        scratch_types=[
            pltpu.VMEM_SHARED((NP,), jnp.float32),
            pltpu.VMEM_SHARED((NP, D), jnp.float32),
            pltpu.VMEM((NP,), jnp.float32),
            pltpu.VMEM((2, _SW, DH), jnp.int32),
            pltpu.VMEM((2, _SW, D), jnp.float32),
            pltpu.VMEM((2, _CHW, _SW), jnp.int32),
            pltpu.VMEM((2, _CHW, _SW), jnp.int32),
            pltpu.VMEM((2, _CHW, _SW), jnp.float32),
            pltpu.VMEM((_SW,), jnp.float32),
            pltpu.SemaphoreType.DMA,
            pltpu.SemaphoreType.DMA,
            pltpu.SemaphoreType.DMA,
            pltpu.SemaphoreType.DMA,
            pltpu.SemaphoreType.DMA,
            pltpu.SemaphoreType.DMA,
        ],
        compiler_params=_sc_params(),
    )
    def k(xq_hbm, s2_hbm, d2_hbm, w2_hbm, p_hbm, deg_hbm,
          deg_sh, agg_sh, isd_l, rows_in, rows_out,
          s_chunk2, d_chunk2, w_chunk2, cbuf,
          gsem0, gsem1, tsem0, tsem1, stsem, psem):
        c = lax.axis_index("c")
        s = lax.axis_index("s")
        wid = s * _NC + c
        zero16 = jnp.zeros((_L,), jnp.float32)
        gsem = (gsem0, gsem1)
        tsem = (tsem0, tsem1)

        @pl.loop(0, _SW)
        def _(r):
            for j in range(D // _L):
                rows_out[0, r, pl.ds(j * _L, _L)] = zero16

        for kk in range(_SW // _L):
            cbuf[pl.ds(kk * _L, _L)] = zero16

        for t in range(RPS // _SW):
            pltpu.sync_copy(rows_out.at[0],
                            agg_sh.at[pl.ds(s * RPS + t * _SW, _SW), :])
            pltpu.sync_copy(cbuf, deg_sh.at[pl.ds(s * RPS + t * _SW, _SW)])
        plsc.subcore_barrier()

        start1 = s * W1
        pltpu.sync_copy(d2_hbm.at[pl.ds(start1, _CHW), :], d_chunk2.at[0])
        pltpu.sync_copy(w2_hbm.at[pl.ds(start1, _CHW), :], w_chunk2.at[0])

        @pl.loop(0, NCH1)
        def _(cki):
            cs = cki % 2
            ns = (cki + 1) % 2
            w0n = start1 + (cki + 1) * _CHW

            @pl.when(cki < NCH1 - 1)
            def _():
                pltpu.async_copy(d2_hbm.at[pl.ds(w0n, _CHW), :],
                                 d_chunk2.at[ns], stsem)
                pltpu.async_copy(w2_hbm.at[pl.ds(w0n, _CHW), :],
                                 w_chunk2.at[ns], stsem)

            descs = [
                pltpu.async_copy(w_chunk2.at[cs].at[j],
                                 deg_sh.at[d_chunk2.at[cs].at[j]],
                                 psem, add=True)
                for j in range(_CHW)
            ]
            for dsc in descs:
                dsc.wait()

            @pl.when(cki < NCH1 - 1)
            def _():
                pltpu.make_async_copy(d2_hbm.at[pl.ds(w0n, _CHW), :],
                                      d_chunk2.at[ns], stsem).wait()
                pltpu.make_async_copy(w2_hbm.at[pl.ds(w0n, _CHW), :],
                                      w_chunk2.at[ns], stsem).wait()

        plsc.subcore_barrier()

        pltpu.sync_copy(deg_sh, isd_l)

        @pl.loop(0, NP // _L)
        def _(t):
            d = isd_l[pl.ds(t * _L, _L)] + 1.0
            i = plsc.bitcast(d, jnp.int32)
            y = plsc.bitcast(jnp.int32(0x5F3759DF) - (i >> 1), jnp.float32)
            y = y * (1.5 - 0.5 * d * y * y)
            y = y * (1.5 - 0.5 * d * y * y)
            y = y * (1.5 - 0.5 * d * y * y)
            isd_l[pl.ds(t * _L, _L)] = y

        @pl.when(c == 0)
        def _():
            pltpu.sync_copy(deg_sh.at[pl.ds(s * RPS, RPS)],
                            deg_hbm.at[pl.ds(s * RPS, RPS)])

        start3 = wid * W3
        himask = jnp.full((_L,), jnp.int32(-65536))

        def scale_window(cs, j, b):
            for kk in range(_SW // _L):
                s16 = s_chunk2[cs, j, pl.ds(kk * _L, _L)]
                isd_s = plsc.load_gather(isd_l, [s16])
                cbuf[pl.ds(kk * _L, _L)] = (
                    w_chunk2[cs, j, pl.ds(kk * _L, _L)] * isd_s)

            @pl.loop(0, _SW // _L)
            def _(g):
                c16 = cbuf[pl.ds(g * _L, _L)]
                for l in range(_L):
                    ce = c16[l]
                    e = g * _L + l
                    for jj in range(DB):
                        word = rows_in[b, e, pl.ds(jj * _L, _L)]
                        lo = plsc.bitcast(word << 16, jnp.float32)
                        hi = plsc.bitcast(word & himask, jnp.float32)
                        rows_out[b, e, pl.ds(jj * 2 * _L, _L)] = lo * ce
                        rows_out[b, e, pl.ds((jj * 2 + 1) * _L, _L)] = hi * ce

        pltpu.sync_copy(s2_hbm.at[pl.ds(start3, _CHW), :], s_chunk2.at[0])
        pltpu.sync_copy(d2_hbm.at[pl.ds(start3, _CHW), :], d_chunk2.at[0])
        pltpu.sync_copy(w2_hbm.at[pl.ds(start3, _CHW), :], w_chunk2.at[0])
        pltpu.async_copy(xq_hbm.at[s_chunk2.at[0].at[0]], rows_in.at[0], gsem[0])

        @pl.loop(0, NCH3)
        def _(cki):
            cs = cki % 2
            ns = (cki + 1) % 2
            w0n = start3 + (cki + 1) * _CHW
            st = []
            for j in range(_CHW):
                b = j % 2
                nb = (j + 1) % 2
                if j == 0:
                    pltpu.async_copy(xq_hbm.at[s_chunk2.at[cs].at[j + 1]],
                                     rows_in.at[nb], gsem[nb])

                    @pl.when(cki < NCH3 - 1)
                    def _():
                        st.append(pltpu.async_copy(
                            s2_hbm.at[pl.ds(w0n, _CHW), :], s_chunk2.at[ns],
                            stsem))
                        st.append(pltpu.async_copy(
                            d2_hbm.at[pl.ds(w0n, _CHW), :], d_chunk2.at[ns],
                            stsem))
                        st.append(pltpu.async_copy(
                            w2_hbm.at[pl.ds(w0n, _CHW), :], w_chunk2.at[ns],
                            stsem))
                elif j < _CHW - 1:
                    pltpu.async_copy(xq_hbm.at[s_chunk2.at[cs].at[j + 1]],
                                     rows_in.at[nb], gsem[nb])
                else:
                    @pl.when(cki < NCH3 - 1)
                    def _():
                        for dsc in st:
                            dsc.wait()
                        pltpu.async_copy(xq_hbm.at[s_chunk2.at[ns].at[0]],
                                         rows_in.at[nb], gsem[nb])
                pltpu.make_async_copy(xq_hbm.at[pl.ds(0, _SW), :],
                                      rows_in.at[b], gsem[b]).wait()
                if j < 2:
                    @pl.when(cki > 0)
                    def _():
                        pltpu.make_async_copy(
                            p_hbm.at[0].at[pl.ds(0, _SW), :], rows_out.at[b],
                            tsem[b]).wait()
                else:
                    pltpu.make_async_copy(p_hbm.at[0].at[pl.ds(0, _SW), :],
                                          rows_out.at[b], tsem[b]).wait()
                scale_window(cs, j, b)
                pltpu.async_copy(rows_out.at[b],
                                 agg_sh.at[d_chunk2.at[cs].at[j]],
                                 tsem[b], add=True)

        pltpu.make_async_copy(p_hbm.at[0].at[pl.ds(0, _SW), :], rows_out.at[0],
                              tsem[0]).wait()
        pltpu.make_async_copy(p_hbm.at[0].at[pl.ds(0, _SW), :], rows_out.at[1],
                              tsem[1]).wait()

        plsc.subcore_barrier()

        for t in range(RPS // _SW):
            sl = pl.ds(s * RPS + t * _SW, _SW)
            pltpu.sync_copy(agg_sh.at[sl, :], p_hbm.at[c].at[sl, :])

    return k(xq, src2, dst2, w2)


def _tc_finish(P, x, deg2, W, b2):
    N, D = x.shape
    RB = 2000
    assert N % RB == 0

    def body(p0_r, p1_r, x_r, deg_r, w_r, b_r, o_r):
        deg = deg_r[...] + 1.0
        agg = lax.rsqrt(deg) * (p0_r[0] + p1_r[0]) + x_r[...] / deg
        y = jnp.dot(agg, w_r[...], preferred_element_type=jnp.float32) + b_r[...]
        o_r[...] = jnp.maximum(y, 0.0)

    return pl.pallas_call(
        body,
        grid=(N // RB,),
        in_specs=[
            pl.BlockSpec((1, RB, D), lambda i: (0, i, 0)),
            pl.BlockSpec((1, RB, D), lambda i: (1, i, 0)),
            pl.BlockSpec((RB, D), lambda i: (i, 0)),
            pl.BlockSpec((RB, 1), lambda i: (i, 0)),
            pl.BlockSpec((D, D), lambda i: (0, 0)),
            pl.BlockSpec((1, D), lambda i: (0, 0)),
        ],
        out_specs=pl.BlockSpec((RB, D), lambda i: (i, 0)),
        out_shape=jax.ShapeDtypeStruct((N, D), jnp.float32),
    )(P, P, x, deg2, W, b2)


def kernel(x, edge_index, edge_weights, W, b):
    N, D = x.shape
    E = edge_index.shape[1]
    NP = 10240
    xb = x.astype(jnp.bfloat16).reshape(N, D // 32, 2, 16)
    xi = jnp.stack([xb[:, :, 0, :], xb[:, :, 1, :]], axis=-1)
    xq = jax.lax.bitcast_convert_type(
        xi.reshape(N, D // 2, 2), jnp.int32)
    unit = _SW * _NC * _NS * _CHW
    EP = -(-E // unit) * unit
    pad = EP - E
    pad_idx = jnp.arange(pad, dtype=jnp.int32) % jnp.int32(N)
    src2 = jnp.concatenate([edge_index[0], pad_idx]).reshape(EP // _SW, _SW)
    dst2 = jnp.concatenate([edge_index[1], pad_idx]).reshape(EP // _SW, _SW)
    w2 = jnp.concatenate(
        [edge_weights, jnp.zeros((pad,), jnp.float32)]).reshape(EP // _SW, _SW)
    P, deg_raw = _sc_aggregate(xq, src2, dst2, w2, NP, D)
    deg2 = deg_raw[:N].reshape(N, 1)
    b2 = b.reshape(1, D)
    return _tc_finish(P, x, deg2, W, b2)


# speedup vs baseline: 1.5282x; 1.5282x over previous
import dataclasses
import functools

import jax
import jax.numpy as jnp
from jax import lax
from jax.experimental import pallas as pl
from jax.experimental.pallas import tpu as pltpu
from jax.experimental.pallas import tpu_sc as plsc

_NC = 2
_NS = 16
_L = 16
_SW = 64
_CHW = 8


def _sc_params():
    cp = pltpu.CompilerParams()
    if "needs_layout_passes" in pltpu.CompilerParams.__dataclass_fields__:
        cp = dataclasses.replace(cp, needs_layout_passes=False)
    return cp


def _sc_aggregate(xq, src2, dst2, w2, n_pad, n_feat):
    D = n_feat
    WN = src2.shape[0]
    NP = n_pad
    NPA = NP - 128
    RPS = NP // _NS
    RPSA = NPA // _NS
    NW = _NC * _NS
    DH = D // 2
    DB = D // 32

    W3 = WN // NW
    W1 = WN // _NS
    NCH3 = W3 // _CHW
    NCH1 = W1 // _CHW
    assert WN % NW == 0 and W3 % _CHW == 0 and W1 % _CHW == 0
    assert RPS % _SW == 0 and RPSA % 8 == 0
    row_chunks = []
    off = 0
    while off < RPSA:
        nn = min(_SW, RPSA - off)
        row_chunks.append((off, nn))
        off += nn

    mesh = plsc.VectorSubcoreMesh(core_axis_name="c", subcore_axis_name="s")

    @functools.partial(
        pl.kernel,
        out_type=(
            jax.ShapeDtypeStruct((_NC, NPA, D), jnp.float32),
            jax.ShapeDtypeStruct((NP,), jnp.float32),
        ),
        mesh=mesh,
        scratch_types=[
            pltpu.VMEM_SHARED((NP,), jnp.float32),
            pltpu.VMEM_SHARED((NPA, D), jnp.float32),
            pltpu.VMEM((NP,), jnp.float32),
            pltpu.VMEM((4, _SW, D), jnp.float32),
            pltpu.VMEM((2, _CHW, _SW), jnp.int32),
            pltpu.VMEM((2, _CHW, _SW), jnp.int32),
            pltpu.VMEM((2, _CHW, _SW), jnp.float32),
            pltpu.VMEM((_SW,), jnp.float32),
            pltpu.SemaphoreType.DMA,
            pltpu.SemaphoreType.DMA,
            pltpu.SemaphoreType.DMA,
            pltpu.SemaphoreType.DMA,
            pltpu.SemaphoreType.DMA,
            pltpu.SemaphoreType.DMA,
            pltpu.SemaphoreType.DMA,
            pltpu.SemaphoreType.DMA,
            pltpu.SemaphoreType.DMA,
            pltpu.SemaphoreType.DMA,
        ],
        compiler_params=_sc_params(),
    )
    def k(xq_hbm, s2_hbm, d2_hbm, w2_hbm, p_hbm, deg_hbm,
          deg_sh, agg_sh, isd_l, rows,
          s_chunk2, d_chunk2, w_chunk2, cbuf,
          gsem0, gsem1, gsem2, gsem3, tsem0, tsem1, tsem2, tsem3,
          stsem, psem):
        c = lax.axis_index("c")
        s = lax.axis_index("s")
        wid = s * _NC + c
        zero16 = jnp.zeros((_L,), jnp.float32)
        gsem = (gsem0, gsem1, gsem2, gsem3)
        tsem = (tsem0, tsem1, tsem2, tsem3)

        @pl.loop(0, _SW)
        def _(r):
            for j in range(D // _L):
                rows[0, r, pl.ds(j * _L, _L)] = zero16

        for kk in range(_SW // _L):
            cbuf[pl.ds(kk * _L, _L)] = zero16

        for off, nn in row_chunks:
            pltpu.sync_copy(rows.at[0].at[pl.ds(0, nn), :],
                            agg_sh.at[pl.ds(s * RPSA + off, nn), :])
        for t in range(RPS // _SW):
            pltpu.sync_copy(cbuf, deg_sh.at[pl.ds(s * RPS + t * _SW, _SW)])
        plsc.subcore_barrier()

        start1 = s * W1
        pltpu.sync_copy(d2_hbm.at[pl.ds(start1, _CHW), :], d_chunk2.at[0])
        pltpu.sync_copy(w2_hbm.at[pl.ds(start1, _CHW), :], w_chunk2.at[0])

        @pl.loop(0, NCH1)
        def _(cki):
            cs = cki % 2
            ns = (cki + 1) % 2
            w0n = start1 + (cki + 1) * _CHW

            @pl.when(cki < NCH1 - 1)
            def _():
                pltpu.async_copy(d2_hbm.at[pl.ds(w0n, _CHW), :],
                                 d_chunk2.at[ns], stsem)
                pltpu.async_copy(w2_hbm.at[pl.ds(w0n, _CHW), :],
                                 w_chunk2.at[ns], stsem)

            descs = [
                pltpu.async_copy(w_chunk2.at[cs].at[j],
                                 deg_sh.at[d_chunk2.at[cs].at[j]],
                                 psem, add=True)
                for j in range(_CHW)
            ]
            for dsc in descs:
                dsc.wait()

            @pl.when(cki < NCH1 - 1)
            def _():
                pltpu.make_async_copy(d2_hbm.at[pl.ds(w0n, _CHW), :],
                                      d_chunk2.at[ns], stsem).wait()
                pltpu.make_async_copy(w2_hbm.at[pl.ds(w0n, _CHW), :],
                                      w_chunk2.at[ns], stsem).wait()

        plsc.subcore_barrier()

        pltpu.sync_copy(deg_sh, isd_l)

        @pl.loop(0, NP // _L)
        def _(t):
            d = isd_l[pl.ds(t * _L, _L)] + 1.0
            i = plsc.bitcast(d, jnp.int32)
            y = plsc.bitcast(jnp.int32(0x5F3759DF) - (i >> 1), jnp.float32)
            y = y * (1.5 - 0.5 * d * y * y)
            y = y * (1.5 - 0.5 * d * y * y)
            y = y * (1.5 - 0.5 * d * y * y)
            isd_l[pl.ds(t * _L, _L)] = y

        @pl.when(c == 0)
        def _():
            pltpu.sync_copy(deg_sh.at[pl.ds(s * RPS, RPS)],
                            deg_hbm.at[pl.ds(s * RPS, RPS)])

        start3 = wid * W3
        himask = jnp.full((_L,), jnp.int32(-65536))

        def scale_window(cs, j, b):
            for kk in range(_SW // _L):
                s16 = s_chunk2[cs, j, pl.ds(kk * _L, _L)]
                isd_s = plsc.load_gather(isd_l, [s16])
                cbuf[pl.ds(kk * _L, _L)] = (
                    w_chunk2[cs, j, pl.ds(kk * _L, _L)] * isd_s)

            @pl.loop(0, _SW // _L)
            def _(g):
                c16 = cbuf[pl.ds(g * _L, _L)]
                for l in range(_L):
                    ce = c16[l]
                    e = g * _L + l
                    for jj in range(D // _L):
                        rows[b, e, pl.ds(jj * _L, _L)] = (
                            rows[b, e, pl.ds(jj * _L, _L)] * ce)

        def drain_t(i):
            pltpu.make_async_copy(p_hbm.at[0].at[pl.ds(0, _SW), :],
                                  rows.at[i], tsem[i]).wait()

        pltpu.sync_copy(s2_hbm.at[pl.ds(start3, _CHW), :], s_chunk2.at[0])
        pltpu.sync_copy(d2_hbm.at[pl.ds(start3, _CHW), :], d_chunk2.at[0])
        pltpu.sync_copy(w2_hbm.at[pl.ds(start3, _CHW), :], w_chunk2.at[0])
        for jp in range(3):
            pltpu.async_copy(xq_hbm.at[s_chunk2.at[0].at[jp]], rows.at[jp],
                             gsem[jp])

        @pl.loop(0, NCH3)
        def _(cki):
            cs = cki % 2
            ns = (cki + 1) % 2
            w0n = start3 + (cki + 1) * _CHW
            st = []
            for j in range(_CHW):
                b = j % 4
                b3 = (j + 3) % 4
                if j + 3 < _CHW:
                    if j == 0:
                        @pl.when(cki > 0)
                        def _():
                            drain_t(b3)
                    else:
                        drain_t(b3)
                    pltpu.async_copy(xq_hbm.at[s_chunk2.at[cs].at[j + 3]],
                                     rows.at[b3], gsem[b3])
                    if j == 0:
                        @pl.when(cki < NCH3 - 1)
                        def _():
                            st.append(pltpu.async_copy(
                                s2_hbm.at[pl.ds(w0n, _CHW), :],
                                s_chunk2.at[ns], stsem))
                            st.append(pltpu.async_copy(
                                d2_hbm.at[pl.ds(w0n, _CHW), :],
                                d_chunk2.at[ns], stsem))
                            st.append(pltpu.async_copy(
                                w2_hbm.at[pl.ds(w0n, _CHW), :],
                                w_chunk2.at[ns], stsem))
                else:
                    @pl.when(cki < NCH3 - 1)
                    def _():
                        if j + 3 == _CHW:
                            for dsc in st:
                                dsc.wait()
                        drain_t(b3)
                        pltpu.async_copy(
                            xq_hbm.at[s_chunk2.at[ns].at[j + 3 - _CHW]],
                            rows.at[b3], gsem[b3])
                pltpu.make_async_copy(xq_hbm.at[pl.ds(0, _SW), :],
                                      rows.at[b], gsem[b]).wait()
                scale_window(cs, j, b)
                pltpu.async_copy(rows.at[b],
                                 agg_sh.at[d_chunk2.at[cs].at[j]],
                                 tsem[b], add=True)

        for i in range(4):
            drain_t(i)

        plsc.subcore_barrier()

        for off, nn in row_chunks:
            sl = pl.ds(s * RPSA + off, nn)
            pltpu.sync_copy(agg_sh.at[sl, :], p_hbm.at[c].at[sl, :])

    return k(xq, src2, dst2, w2)


def _tc_finish(P, x, deg2, W, b2):
    N, D = x.shape
    RB = 2000
    assert N % RB == 0

    def body(p0_r, p1_r, x_r, deg_r, w_r, b_r, o_r):
        deg = deg_r[...] + 1.0
        agg = lax.rsqrt(deg) * (p0_r[0] + p1_r[0]) + x_r[...] / deg
        y = jnp.dot(agg, w_r[...], preferred_element_type=jnp.float32) + b_r[...]
        o_r[...] = jnp.maximum(y, 0.0)

    return pl.pallas_call(
        body,
        grid=(N // RB,),
        in_specs=[
            pl.BlockSpec((1, RB, D), lambda i: (0, i, 0)),
            pl.BlockSpec((1, RB, D), lambda i: (1, i, 0)),
            pl.BlockSpec((RB, D), lambda i: (i, 0)),
            pl.BlockSpec((RB, 1), lambda i: (i, 0)),
            pl.BlockSpec((D, D), lambda i: (0, 0)),
            pl.BlockSpec((1, D), lambda i: (0, 0)),
        ],
        out_specs=pl.BlockSpec((RB, D), lambda i: (i, 0)),
        out_shape=jax.ShapeDtypeStruct((N, D), jnp.float32),
    )(P, P, x, deg2, W, b2)


def kernel(x, edge_index, edge_weights, W, b):
    N, D = x.shape
    E = edge_index.shape[1]
    NP = 10240
    xq = x
    unit = _SW * _NC * _NS * _CHW
    EP = -(-E // unit) * unit
    pad = EP - E
    pad_idx = jnp.arange(pad, dtype=jnp.int32) % jnp.int32(N)
    src2 = jnp.concatenate([edge_index[0], pad_idx]).reshape(EP // _SW, _SW)
    dst2 = jnp.concatenate([edge_index[1], pad_idx]).reshape(EP // _SW, _SW)
    w2 = jnp.concatenate(
        [edge_weights, jnp.zeros((pad,), jnp.float32)]).reshape(EP // _SW, _SW)
    P, deg_raw = _sc_aggregate(xq, src2, dst2, w2, NP, D)
    deg2 = deg_raw[:N].reshape(N, 1)
    b2 = b.reshape(1, D)
    return _tc_finish(P, x, deg2, W, b2)


# speedup vs baseline: 1.7479x; 1.1438x over previous
import dataclasses
import functools

import jax
import jax.numpy as jnp
from jax import lax
from jax.experimental import pallas as pl
from jax.experimental.pallas import tpu as pltpu
from jax.experimental.pallas import tpu_sc as plsc

_NC = 2
_NS = 16
_L = 16
_WIN = 128
_CH = 4


def _sc_params():
    cp = pltpu.CompilerParams()
    if "needs_layout_passes" in pltpu.CompilerParams.__dataclass_fields__:
        cp = dataclasses.replace(cp, needs_layout_passes=False)
    return cp


def _sc_aggregate(x, src2, dst2, w2, n_pad):
    N, D = x.shape
    WN = src2.shape[0]
    NP = n_pad
    RPS = NP // _NS
    NW = _NC * _NS
    DG = D // _L

    W3 = WN // NW
    W1 = WN // _NS
    NCH3 = W3 // _CH
    NCH1 = W1 // _CH
    assert WN % NW == 0 and W3 % _CH == 0 and W1 % _CH == 0

    mesh = plsc.VectorSubcoreMesh(core_axis_name="c", subcore_axis_name="s")

    @functools.partial(
        pl.kernel,
        out_type=(
            jax.ShapeDtypeStruct((_NC, NP, D), jnp.float32),
            jax.ShapeDtypeStruct((NP,), jnp.float32),
        ),
        mesh=mesh,
        scratch_types=[
            pltpu.VMEM_SHARED((NP,), jnp.float32),
            pltpu.VMEM_SHARED((NP, D), jnp.float32),
            pltpu.VMEM((NP,), jnp.float32),
            pltpu.VMEM((2, _WIN, D), jnp.float32),
            pltpu.VMEM((_WIN,), jnp.float32),
            pltpu.VMEM((2, _CH, _WIN), jnp.int32),
            pltpu.VMEM((2, _CH, _WIN), jnp.int32),
            pltpu.VMEM((2, _CH, _WIN), jnp.float32),
            pltpu.VMEM((_WIN,), jnp.float32),
            pltpu.SemaphoreType.DMA,
            pltpu.SemaphoreType.DMA,
            pltpu.SemaphoreType.DMA,
            pltpu.SemaphoreType.DMA,
            pltpu.SemaphoreType.DMA,
            pltpu.SemaphoreType.DMA,
            pltpu.SemaphoreType.DMA,
        ],
        compiler_params=_sc_params(),
    )
    def k(x_hbm, s2_hbm, d2_hbm, w2_hbm, p_hbm, deg_hbm,
          deg_sh, agg_sh, isd_l, rows2, zbuf,
          s_chunk2, d_chunk2, w_chunk2, cbuf,
          gsem0, gsem1, tsem0, tsem1, stsem, psem, qsem):
        c = lax.axis_index("c")
        s = lax.axis_index("s")
        wid = s * _NC + c
        zero16 = jnp.zeros((_L,), jnp.float32)
        gsem = (gsem0, gsem1)
        tsem = (tsem0, tsem1)
        start1 = s * W1
        start3 = wid * W3

        @pl.loop(0, _WIN)
        def _(r):
            for j in range(DG):
                rows2[0, r, pl.ds(j * _L, _L)] = zero16

        @pl.loop(0, _WIN // _L)
        def _(t):
            zbuf[pl.ds(t * _L, _L)] = zero16

        for t in range(RPS // _WIN):
            pltpu.sync_copy(rows2.at[0],
                            agg_sh.at[pl.ds(s * RPS + t * _WIN, _WIN), :])
            pltpu.sync_copy(zbuf, deg_sh.at[pl.ds(s * RPS + t * _WIN, _WIN)])
        plsc.subcore_barrier()

        pltpu.async_copy(s2_hbm.at[pl.ds(start3, _CH), :], s_chunk2.at[0],
                         qsem)

        pltpu.sync_copy(d2_hbm.at[pl.ds(start1, _CH), :], d_chunk2.at[0])
        pltpu.sync_copy(w2_hbm.at[pl.ds(start1, _CH), :], w_chunk2.at[0])

        @pl.loop(0, NCH1)
        def _(cki):
            cs = cki % 2
            ns = (cki + 1) % 2
            w0n = start1 + (cki + 1) * _CH

            @pl.when(cki < NCH1 - 1)
            def _():
                pltpu.async_copy(d2_hbm.at[pl.ds(w0n, _CH), :],
                                 d_chunk2.at[ns], stsem)
                pltpu.async_copy(w2_hbm.at[pl.ds(w0n, _CH), :],
                                 w_chunk2.at[ns], stsem)

            descs = [
                pltpu.async_copy(w_chunk2.at[cs].at[j],
                                 deg_sh.at[d_chunk2.at[cs].at[j]],
                                 psem, add=True)
                for j in range(_CH)
            ]
            for dsc in descs:
                dsc.wait()

            @pl.when(cki < NCH1 - 1)
            def _():
                pltpu.make_async_copy(d2_hbm.at[pl.ds(w0n, _CH), :],
                                      d_chunk2.at[ns], stsem).wait()
                pltpu.make_async_copy(w2_hbm.at[pl.ds(w0n, _CH), :],
                                      w_chunk2.at[ns], stsem).wait()

        plsc.subcore_barrier()

        pltpu.sync_copy(deg_sh, isd_l)

        @pl.loop(0, NP // (2 * _L))
        def _(t):
            for h in range(2):
                d = isd_l[pl.ds((2 * t + h) * _L, _L)] + 1.0
                i = plsc.bitcast(d, jnp.int32)
                y = plsc.bitcast(jnp.int32(0x5F3759DF) - (i >> 1), jnp.float32)
                y = y * (1.5 - 0.5 * d * y * y)
                y = y * (1.5 - 0.5 * d * y * y)
                y = y * (1.5 - 0.5 * d * y * y)
                isd_l[pl.ds((2 * t + h) * _L, _L)] = y

        @pl.when(c == 0)
        def _():
            pltpu.sync_copy(deg_sh.at[pl.ds(s * RPS, RPS)],
                            deg_hbm.at[pl.ds(s * RPS, RPS)])

        def scale_window(cs, j, b):
            for kk in range(_WIN // _L):
                s16 = s_chunk2[cs, j, pl.ds(kk * _L, _L)]
                isd_s = plsc.load_gather(isd_l, [s16])
                cbuf[pl.ds(kk * _L, _L)] = (
                    w_chunk2[cs, j, pl.ds(kk * _L, _L)] * isd_s)

            @pl.loop(0, _WIN // _L)
            def _(g):
                c16 = cbuf[pl.ds(g * _L, _L)]
                for l in range(_L):
                    ce = c16[l]
                    e = g * _L + l
                    for jj in range(DG):
                        rows2[b, e, pl.ds(jj * _L, _L)] = (
                            rows2[b, e, pl.ds(jj * _L, _L)] * ce)

        pltpu.make_async_copy(s2_hbm.at[pl.ds(start3, _CH), :], s_chunk2.at[0],
                              qsem).wait()
        pltpu.async_copy(x_hbm.at[s_chunk2.at[0].at[0]], rows2.at[0], gsem[0])
        pltpu.sync_copy(d2_hbm.at[pl.ds(start3, _CH), :], d_chunk2.at[0])
        pltpu.sync_copy(w2_hbm.at[pl.ds(start3, _CH), :], w_chunk2.at[0])

        @pl.loop(0, NCH3)
        def _(cki):
            cs = cki % 2
            ns = (cki + 1) % 2
            w0n = start3 + (cki + 1) * _CH
            st = []
            for j in range(_CH):
                b = j % 2
                nb = (j + 1) % 2
                if j == 0:
                    @pl.when(cki > 0)
                    def _():
                        pltpu.make_async_copy(
                            x_hbm.at[pl.ds(0, _WIN), :], rows2.at[nb],
                            tsem[nb]).wait()
                    pltpu.async_copy(x_hbm.at[s_chunk2.at[cs].at[j + 1]],
                                     rows2.at[nb], gsem[nb])

                    @pl.when(cki < NCH3 - 1)
                    def _():
                        st.append(pltpu.async_copy(
                            s2_hbm.at[pl.ds(w0n, _CH), :], s_chunk2.at[ns],
                            stsem))
                        st.append(pltpu.async_copy(
                            d2_hbm.at[pl.ds(w0n, _CH), :], d_chunk2.at[ns],
                            stsem))
                        st.append(pltpu.async_copy(
                            w2_hbm.at[pl.ds(w0n, _CH), :], w_chunk2.at[ns],
                            stsem))
                elif j < _CH - 1:
                    pltpu.make_async_copy(x_hbm.at[pl.ds(0, _WIN), :],
                                          rows2.at[nb], tsem[nb]).wait()
                    pltpu.async_copy(x_hbm.at[s_chunk2.at[cs].at[j + 1]],
                                     rows2.at[nb], gsem[nb])
                else:
                    @pl.when(cki < NCH3 - 1)
                    def _():
                        for dsc in st:
                            dsc.wait()
                        pltpu.make_async_copy(x_hbm.at[pl.ds(0, _WIN), :],
                                              rows2.at[nb], tsem[nb]).wait()
                        pltpu.async_copy(x_hbm.at[s_chunk2.at[ns].at[0]],
                                         rows2.at[nb], gsem[nb])
                pltpu.make_async_copy(x_hbm.at[pl.ds(0, _WIN), :],
                                      rows2.at[b], gsem[b]).wait()
                scale_window(cs, j, b)
                pltpu.async_copy(rows2.at[b], agg_sh.at[d_chunk2.at[cs].at[j]],
                                 tsem[b], add=True)

        pltpu.make_async_copy(x_hbm.at[pl.ds(0, _WIN), :], rows2.at[0],
                              tsem[0]).wait()
        pltpu.make_async_copy(x_hbm.at[pl.ds(0, _WIN), :], rows2.at[1],
                              tsem[1]).wait()

        plsc.subcore_barrier()

        for t in range(RPS // _WIN):
            sl = pl.ds(s * RPS + t * _WIN, _WIN)
            pltpu.sync_copy(agg_sh.at[sl, :], p_hbm.at[c].at[sl, :])

    return k(x, src2, dst2, w2)


def _tc_finish(P, x, deg2, W, b2):
    N, D = x.shape
    RB = 2000
    assert N % RB == 0

    def body(p0_r, p1_r, x_r, deg_r, w_r, b_r, o_r):
        deg = deg_r[...] + 1.0
        agg = lax.rsqrt(deg) * (p0_r[0] + p1_r[0]) + x_r[...] / deg
        y = jnp.dot(agg, w_r[...], preferred_element_type=jnp.float32) + b_r[...]
        o_r[...] = jnp.maximum(y, 0.0)

    return pl.pallas_call(
        body,
        grid=(N // RB,),
        in_specs=[
            pl.BlockSpec((1, RB, D), lambda i: (0, i, 0)),
            pl.BlockSpec((1, RB, D), lambda i: (1, i, 0)),
            pl.BlockSpec((RB, D), lambda i: (i, 0)),
            pl.BlockSpec((RB, 1), lambda i: (i, 0)),
            pl.BlockSpec((D, D), lambda i: (0, 0)),
            pl.BlockSpec((1, D), lambda i: (0, 0)),
        ],
        out_specs=pl.BlockSpec((RB, D), lambda i: (i, 0)),
        out_shape=jax.ShapeDtypeStruct((N, D), jnp.float32),
    )(P, P, x, deg2, W, b2)


def kernel(x, edge_index, edge_weights, W, b):
    N, D = x.shape
    E = edge_index.shape[1]
    NP = 10240
    unit = _WIN * _NC * _NS * _CH
    EP = -(-E // unit) * unit
    pad = EP - E
    pad_idx = jnp.arange(pad, dtype=jnp.int32) % jnp.int32(N)
    src2 = jnp.concatenate([edge_index[0], pad_idx]).reshape(EP // _WIN, _WIN)
    dst2 = jnp.concatenate([edge_index[1], pad_idx]).reshape(EP // _WIN, _WIN)
    w2 = jnp.concatenate(
        [edge_weights, jnp.zeros((pad,), jnp.float32)]).reshape(EP // _WIN, _WIN)
    P, deg_raw = _sc_aggregate(x, src2, dst2, w2, NP)
    deg2 = deg_raw[:N].reshape(N, 1)
    b2 = b.reshape(1, D)
    return _tc_finish(P, x, deg2, W, b2)


# speedup vs baseline: 1.7722x; 1.0139x over previous
import dataclasses
import functools

import jax
import jax.numpy as jnp
from jax import lax
from jax.experimental import pallas as pl
from jax.experimental.pallas import tpu as pltpu
from jax.experimental.pallas import tpu_sc as plsc

_NC = 2
_NS = 16
_L = 16
_WIN = 128
_CH = 4


def _sc_params():
    cp = pltpu.CompilerParams()
    if "needs_layout_passes" in pltpu.CompilerParams.__dataclass_fields__:
        cp = dataclasses.replace(cp, needs_layout_passes=False)
    return cp


def _sc_aggregate(x, src2, dst2, w2, n_pad):
    N, D = x.shape
    WN = src2.shape[0]
    NP = n_pad
    RPS = NP // _NS
    NW = _NC * _NS
    DG = D // _L

    W3 = WN // NW
    W1 = WN // _NS
    NCH3 = W3 // _CH
    NCH1 = W1 // _CH
    assert WN % NW == 0 and W3 % _CH == 0 and W1 % _CH == 0

    mesh = plsc.VectorSubcoreMesh(core_axis_name="c", subcore_axis_name="s")

    @functools.partial(
        pl.kernel,
        out_type=(
            jax.ShapeDtypeStruct((_NC, NP, D), jnp.float32),
            jax.ShapeDtypeStruct((NP,), jnp.float32),
        ),
        mesh=mesh,
        scratch_types=[
            pltpu.VMEM_SHARED((NP,), jnp.float32),
            pltpu.VMEM_SHARED((NP, D), jnp.float32),
            pltpu.VMEM((NP,), jnp.float32),
            pltpu.VMEM((2, _WIN, D), jnp.float32),
            pltpu.VMEM((_WIN,), jnp.float32),
            pltpu.VMEM((2, _CH, _WIN), jnp.int32),
            pltpu.VMEM((2, _CH, _WIN), jnp.int32),
            pltpu.VMEM((2, _CH, _WIN), jnp.float32),
            pltpu.VMEM((_WIN,), jnp.float32),
            pltpu.SemaphoreType.DMA,
            pltpu.SemaphoreType.DMA,
            pltpu.SemaphoreType.DMA,
            pltpu.SemaphoreType.DMA,
            pltpu.SemaphoreType.DMA,
            pltpu.SemaphoreType.DMA,
            pltpu.SemaphoreType.DMA,
            pltpu.SemaphoreType.DMA,
        ],
        compiler_params=_sc_params(),
    )
    def k(x_hbm, s2_hbm, d2_hbm, w2_hbm, p_hbm, deg_hbm,
          deg_sh, agg_sh, isd_l, rows2, zbuf,
          s_chunk2, d_chunk2, w_chunk2, cbuf,
          gsem0, gsem1, tsem0, tsem1, stsem, psem, qsem, zsem):
        c = lax.axis_index("c")
        s = lax.axis_index("s")
        wid = s * _NC + c
        zero16 = jnp.zeros((_L,), jnp.float32)
        gsem = (gsem0, gsem1)
        tsem = (tsem0, tsem1)
        start1 = s * W1
        start3 = wid * W3

        @pl.loop(0, _WIN)
        def _(r):
            for j in range(DG):
                rows2[0, r, pl.ds(j * _L, _L)] = zero16

        @pl.loop(0, _WIN // _L)
        def _(t):
            zbuf[pl.ds(t * _L, _L)] = zero16

        for t in range(RPS // _WIN):
            pltpu.sync_copy(zbuf, deg_sh.at[pl.ds(s * RPS + t * _WIN, _WIN)])
        plsc.subcore_barrier()

        zdescs = [
            pltpu.async_copy(rows2.at[0],
                             agg_sh.at[pl.ds(s * RPS + t * _WIN, _WIN), :],
                             zsem)
            for t in range(RPS // _WIN)
        ]

        pltpu.async_copy(s2_hbm.at[pl.ds(start3, _CH), :], s_chunk2.at[0],
                         qsem)

        pltpu.sync_copy(d2_hbm.at[pl.ds(start1, _CH), :], d_chunk2.at[0])
        pltpu.sync_copy(w2_hbm.at[pl.ds(start1, _CH), :], w_chunk2.at[0])

        @pl.loop(0, NCH1)
        def _(cki):
            cs = cki % 2
            ns = (cki + 1) % 2
            w0n = start1 + (cki + 1) * _CH

            @pl.when(cki > 0)
            def _():
                for j in range(_CH):
                    pltpu.make_async_copy(
                        w_chunk2.at[ns].at[j],
                        deg_sh.at[pl.ds(0, _WIN)], psem).wait()

            @pl.when(cki > 0)
            def _():
                pltpu.make_async_copy(d2_hbm.at[pl.ds(w0n, _CH), :],
                                      d_chunk2.at[cs], stsem).wait()
                pltpu.make_async_copy(w2_hbm.at[pl.ds(w0n, _CH), :],
                                      w_chunk2.at[cs], stsem).wait()

            @pl.when(cki < NCH1 - 1)
            def _():
                pltpu.async_copy(d2_hbm.at[pl.ds(w0n, _CH), :],
                                 d_chunk2.at[ns], stsem)
                pltpu.async_copy(w2_hbm.at[pl.ds(w0n, _CH), :],
                                 w_chunk2.at[ns], stsem)

            for j in range(_CH):
                pltpu.async_copy(w_chunk2.at[cs].at[j],
                                 deg_sh.at[d_chunk2.at[cs].at[j]],
                                 psem, add=True)

        for j in range(_CH):
            pltpu.make_async_copy(w_chunk2.at[0].at[j],
                                  deg_sh.at[pl.ds(0, _WIN)], psem).wait()
        for dsc in zdescs:
            dsc.wait()
        plsc.subcore_barrier()

        pltpu.sync_copy(deg_sh, isd_l)

        @pl.loop(0, NP // (2 * _L))
        def _(t):
            for h in range(2):
                d = isd_l[pl.ds((2 * t + h) * _L, _L)] + 1.0
                i = plsc.bitcast(d, jnp.int32)
                y = plsc.bitcast(jnp.int32(0x5F3759DF) - (i >> 1), jnp.float32)
                y = y * (1.5 - 0.5 * d * y * y)
                y = y * (1.5 - 0.5 * d * y * y)
                y = y * (1.5 - 0.5 * d * y * y)
                isd_l[pl.ds((2 * t + h) * _L, _L)] = y

        @pl.when(c == 0)
        def _():
            pltpu.sync_copy(deg_sh.at[pl.ds(s * RPS, RPS)],
                            deg_hbm.at[pl.ds(s * RPS, RPS)])

        def scale_window(cs, j, b):
            for kk in range(_WIN // _L):
                s16 = s_chunk2[cs, j, pl.ds(kk * _L, _L)]
                isd_s = plsc.load_gather(isd_l, [s16])
                cbuf[pl.ds(kk * _L, _L)] = (
                    w_chunk2[cs, j, pl.ds(kk * _L, _L)] * isd_s)

            @pl.loop(0, _WIN // _L)
            def _(g):
                c16 = cbuf[pl.ds(g * _L, _L)]
                for l in range(_L):
                    ce = c16[l]
                    e = g * _L + l
                    for jj in range(DG):
                        rows2[b, e, pl.ds(jj * _L, _L)] = (
                            rows2[b, e, pl.ds(jj * _L, _L)] * ce)

        pltpu.make_async_copy(s2_hbm.at[pl.ds(start3, _CH), :], s_chunk2.at[0],
                              qsem).wait()
        pltpu.async_copy(x_hbm.at[s_chunk2.at[0].at[0]], rows2.at[0], gsem[0])
        pltpu.sync_copy(d2_hbm.at[pl.ds(start3, _CH), :], d_chunk2.at[0])
        pltpu.sync_copy(w2_hbm.at[pl.ds(start3, _CH), :], w_chunk2.at[0])

        @pl.loop(0, NCH3)
        def _(cki):
            cs = cki % 2
            ns = (cki + 1) % 2
            w0n = start3 + (cki + 1) * _CH
            st = []
            for j in range(_CH):
                b = j % 2
                nb = (j + 1) % 2
                if j == 0:
                    @pl.when(cki > 0)
                    def _():
                        pltpu.make_async_copy(
                            x_hbm.at[pl.ds(0, _WIN), :], rows2.at[nb],
                            tsem[nb]).wait()
                    pltpu.async_copy(x_hbm.at[s_chunk2.at[cs].at[j + 1]],
                                     rows2.at[nb], gsem[nb])

                    @pl.when(cki < NCH3 - 1)
                    def _():
                        st.append(pltpu.async_copy(
                            s2_hbm.at[pl.ds(w0n, _CH), :], s_chunk2.at[ns],
                            stsem))
                        st.append(pltpu.async_copy(
                            d2_hbm.at[pl.ds(w0n, _CH), :], d_chunk2.at[ns],
                            stsem))
                        st.append(pltpu.async_copy(
                            w2_hbm.at[pl.ds(w0n, _CH), :], w_chunk2.at[ns],
                            stsem))
                elif j < _CH - 1:
                    pltpu.make_async_copy(x_hbm.at[pl.ds(0, _WIN), :],
                                          rows2.at[nb], tsem[nb]).wait()
                    pltpu.async_copy(x_hbm.at[s_chunk2.at[cs].at[j + 1]],
                                     rows2.at[nb], gsem[nb])
                else:
                    @pl.when(cki < NCH3 - 1)
                    def _():
                        for dsc in st:
                            dsc.wait()
                        pltpu.make_async_copy(x_hbm.at[pl.ds(0, _WIN), :],
                                              rows2.at[nb], tsem[nb]).wait()
                        pltpu.async_copy(x_hbm.at[s_chunk2.at[ns].at[0]],
                                         rows2.at[nb], gsem[nb])
                pltpu.make_async_copy(x_hbm.at[pl.ds(0, _WIN), :],
                                      rows2.at[b], gsem[b]).wait()
                scale_window(cs, j, b)
                pltpu.async_copy(rows2.at[b], agg_sh.at[d_chunk2.at[cs].at[j]],
                                 tsem[b], add=True)

        pltpu.make_async_copy(x_hbm.at[pl.ds(0, _WIN), :], rows2.at[0],
                              tsem[0]).wait()
        pltpu.make_async_copy(x_hbm.at[pl.ds(0, _WIN), :], rows2.at[1],
                              tsem[1]).wait()

        plsc.subcore_barrier()

        for t in range(RPS // _WIN):
            sl = pl.ds(s * RPS + t * _WIN, _WIN)
            pltpu.sync_copy(agg_sh.at[sl, :], p_hbm.at[c].at[sl, :])

    return k(x, src2, dst2, w2)


def _tc_finish(P, x, deg2, W, b2):
    N, D = x.shape
    RB = 2000
    assert N % RB == 0

    def body(p0_r, p1_r, x_r, deg_r, w_r, b_r, o_r):
        deg = deg_r[...] + 1.0
        agg = lax.rsqrt(deg) * (p0_r[0] + p1_r[0]) + x_r[...] / deg
        y = jnp.dot(agg, w_r[...], preferred_element_type=jnp.float32) + b_r[...]
        o_r[...] = jnp.maximum(y, 0.0)

    return pl.pallas_call(
        body,
        grid=(N // RB,),
        in_specs=[
            pl.BlockSpec((1, RB, D), lambda i: (0, i, 0)),
            pl.BlockSpec((1, RB, D), lambda i: (1, i, 0)),
            pl.BlockSpec((RB, D), lambda i: (i, 0)),
            pl.BlockSpec((RB, 1), lambda i: (i, 0)),
            pl.BlockSpec((D, D), lambda i: (0, 0)),
            pl.BlockSpec((1, D), lambda i: (0, 0)),
        ],
        out_specs=pl.BlockSpec((RB, D), lambda i: (i, 0)),
        out_shape=jax.ShapeDtypeStruct((N, D), jnp.float32),
    )(P, P, x, deg2, W, b2)


def kernel(x, edge_index, edge_weights, W, b):
    N, D = x.shape
    E = edge_index.shape[1]
    NP = 10240
    unit = _WIN * _NC * _NS * _CH
    EP = -(-E // unit) * unit
    pad = EP - E
    pad_idx = jnp.arange(pad, dtype=jnp.int32) % jnp.int32(N)
    src2 = jnp.concatenate([edge_index[0], pad_idx]).reshape(EP // _WIN, _WIN)
    dst2 = jnp.concatenate([edge_index[1], pad_idx]).reshape(EP // _WIN, _WIN)
    w2 = jnp.concatenate(
        [edge_weights, jnp.zeros((pad,), jnp.float32)]).reshape(EP // _WIN, _WIN)
    P, deg_raw = _sc_aggregate(x, src2, dst2, w2, NP)
    deg2 = deg_raw[:N].reshape(N, 1)
    b2 = b.reshape(1, D)
    return _tc_finish(P, x, deg2, W, b2)
